# baseline (speedup 1.0000x reference)
import jax
import jax.numpy as jnp
from jax import lax
from jax.experimental import pallas as pl
from jax.experimental.pallas import tpu as pltpu
from jax.experimental.pallas import tpu_sc as plsc

BATCH = 16384
EMBED = 32
NODE = 1000000
TILE_ROWS = 8
NUM_CORES = 2
NUM_SUBCORES = 16
NUM_WORKERS = NUM_CORES * NUM_SUBCORES
B_PER_W = BATCH // NUM_WORKERS
IDX_ROWS = 4
CHUNK = 16
NCHUNK = B_PER_W // CHUNK
LN2 = 0.6931471805599453


def _log1p_of_exp_neg(az):
    u = jnp.exp(-az)
    y = 1.0 + u
    bits = plsc.bitcast(y, jnp.int32)
    e = (bits >> 23) - 127
    m = plsc.bitcast((bits & 0x007FFFFF) | 0x3F800000, jnp.float32)
    s = (m - 1.0) / (m + 1.0)
    s2 = s * s
    poly = 1.0 + s2 * (1.0 / 3.0 + s2 * (1.0 / 5.0 + s2 * (1.0 / 7.0 + s2 * (1.0 / 9.0))))
    return e.astype(jnp.float32) * LN2 + 2.0 * s * poly


def _sc_body(a_hbm, b_hbm, sign_hbm, emb_hbm, ctx_hbm, out_hbm,
             a_idx, b_idx, a_rec, b_rec, a_sub, b_sub,
             a_t0, a_t1, b_t0, b_t1, out_v, sign_v, sem0, sem1):
    wid = lax.axis_index("s") * NUM_CORES + lax.axis_index("c")
    base = wid * B_PER_W

    pltpu.sync_copy(a_hbm.at[pl.ds(wid * IDX_ROWS, IDX_ROWS)], a_idx)
    pltpu.sync_copy(b_hbm.at[pl.ds(wid * IDX_ROWS, IDX_ROWS)], b_idx)
    pltpu.sync_copy(sign_hbm, sign_v)

    for j in range(IDX_ROWS):
        for t in range(0, 128, 16):
            va = a_idx[j, pl.ds(t, 16)]
            vb = b_idx[j, pl.ds(t, 16)]
            pos = j * 128 + t
            a_rec[pl.ds(pos, 16)] = va >> 3
            b_rec[pl.ds(pos, 16)] = vb >> 3
            a_sub[pl.ds(pos, 16)] = va & 7
            b_sub[pl.ds(pos, 16)] = vb & 7

    lanes = lax.iota(jnp.int32, 16)
    sign_vec = sign_v[...]

    def fire(c, at, bt, sem):
        va = a_rec[pl.ds(c * CHUNK, 16)]
        vb = b_rec[pl.ds(c * CHUNK, 16)]
        for r in range(16):
            pltpu.async_copy(emb_hbm.at[va[r]], at.at[r], sem)
            pltpu.async_copy(ctx_hbm.at[vb[r]], bt.at[r], sem)

    def drain(at, bt, sem):
        for r in range(16):
            pltpu.make_async_copy(emb_hbm.at[0], at.at[r], sem).wait()
            pltpu.make_async_copy(ctx_hbm.at[0], bt.at[r], sem).wait()

    def compute(c, at, bt):
        pos = c * CHUNK
        sub_a = a_sub[pl.ds(pos, 16)]
        sub_b = b_sub[pl.ds(pos, 16)]
        acc = jnp.zeros((16,), jnp.float32)
        for d in range(EMBED):
            d_vec = jnp.full((16,), d, jnp.int32)
            av = plsc.load_gather(at, [lanes, sub_a, d_vec])
            bv = plsc.load_gather(bt, [lanes, sub_b, d_vec])
            acc = acc + av * bv
        z = -(sign_vec * acc)
        loss = jnp.maximum(z, 0.0) + _log1p_of_exp_neg(jnp.abs(z))
        out_v[pl.ds(pos, 16)] = loss

    fire(0, a_t0, b_t0, sem0)
    fire(1, a_t1, b_t1, sem1)

    def body(i, carry):
        e = i * 2
        drain(a_t0, b_t0, sem0)
        compute(e, a_t0, b_t0)
        fire(e + 2, a_t0, b_t0, sem0)
        drain(a_t1, b_t1, sem1)
        compute(e + 1, a_t1, b_t1)
        fire(e + 3, a_t1, b_t1, sem1)
        return carry

    lax.fori_loop(0, NCHUNK // 2 - 1, body, 0)

    e = NCHUNK - 2
    drain(a_t0, b_t0, sem0)
    compute(e, a_t0, b_t0)
    drain(a_t1, b_t1, sem1)
    compute(e + 1, a_t1, b_t1)

    pltpu.sync_copy(out_v, out_hbm.at[pl.ds(base, B_PER_W)])


def kernel(a, b, sign, embeddings, context_embeddings):
    a2 = a.astype(jnp.int32).reshape(NUM_WORKERS * IDX_ROWS, 128)
    b2 = b.astype(jnp.int32).reshape(NUM_WORKERS * IDX_ROWS, 128)
    one = jnp.exp(jnp.asarray(sign, jnp.float32) * 0.0)
    emb3 = (embeddings * one).reshape(NODE // TILE_ROWS, TILE_ROWS, EMBED)
    ctx3 = (context_embeddings * one).reshape(NODE // TILE_ROWS, TILE_ROWS, EMBED)
    sign_vec = jnp.broadcast_to(jnp.asarray(sign, jnp.float32), (16,))

    buf = pltpu.VMEM((CHUNK, TILE_ROWS, EMBED), jnp.float32)
    mesh = plsc.VectorSubcoreMesh(core_axis_name="c", subcore_axis_name="s")
    run = pl.kernel(
        _sc_body,
        out_type=jax.ShapeDtypeStruct((BATCH,), jnp.float32),
        mesh=mesh,
        compiler_params=pltpu.CompilerParams(needs_layout_passes=False),
        scratch_types=[
            pltpu.VMEM((IDX_ROWS, 128), jnp.int32),
            pltpu.VMEM((IDX_ROWS, 128), jnp.int32),
            pltpu.VMEM((B_PER_W,), jnp.int32),
            pltpu.VMEM((B_PER_W,), jnp.int32),
            pltpu.VMEM((B_PER_W,), jnp.int32),
            pltpu.VMEM((B_PER_W,), jnp.int32),
            buf, buf, buf, buf,
            pltpu.VMEM((B_PER_W,), jnp.float32),
            pltpu.VMEM((16,), jnp.float32),
            pltpu.SemaphoreType.DMA,
            pltpu.SemaphoreType.DMA,
        ],
    )
    return run(a2, b2, sign_vec, emb3, ctx3)


# speedup vs baseline: 2.0375x; 2.0375x over previous
import jax
import jax.numpy as jnp
from jax import lax
from jax.experimental import pallas as pl
from jax.experimental.pallas import tpu as pltpu
from jax.experimental.pallas import tpu_sc as plsc

BATCH = 16384
EMBED = 32
NODE = 1000000
TILE_ROWS = 8
NUM_CORES = 2
NUM_SUBCORES = 16
NUM_WORKERS = NUM_CORES * NUM_SUBCORES
B_PER_W = BATCH // NUM_WORKERS
IDX_ROWS = 4
CHUNK = 16
NCHUNK = B_PER_W // CHUNK
LN2 = 0.6931471805599453


def _log1p_of_exp_neg(az):
    u = jnp.exp(-az)
    y = 1.0 + u
    bits = plsc.bitcast(y, jnp.int32)
    e = (bits >> 23) - 127
    m = plsc.bitcast((bits & 0x007FFFFF) | 0x3F800000, jnp.float32)
    s = (m - 1.0) / (m + 1.0)
    s2 = s * s
    poly = 1.0 + s2 * (1.0 / 3.0 + s2 * (1.0 / 5.0 + s2 * (1.0 / 7.0 + s2 * (1.0 / 9.0))))
    return e.astype(jnp.float32) * LN2 + 2.0 * s * poly


def _sc_body(a_hbm, b_hbm, sign_hbm, emb_hbm, ctx_hbm, out_hbm,
             a_idx, b_idx, a_rec, b_rec, a_sub, b_sub,
             a_t0, a_t1, b_t0, b_t1, out_v, sign_v, sem0, sem1):
    wid = lax.axis_index("s") * NUM_CORES + lax.axis_index("c")
    base = wid * B_PER_W

    pltpu.sync_copy(a_hbm.at[pl.ds(wid * IDX_ROWS, IDX_ROWS)], a_idx)
    pltpu.sync_copy(b_hbm.at[pl.ds(wid * IDX_ROWS, IDX_ROWS)], b_idx)
    pltpu.sync_copy(sign_hbm, sign_v)

    for j in range(IDX_ROWS):
        for t in range(0, 128, 16):
            va = a_idx[j, pl.ds(t, 16)]
            vb = b_idx[j, pl.ds(t, 16)]
            pos = j * 128 + t
            a_rec[pl.ds(pos, 16)] = va >> 3
            b_rec[pl.ds(pos, 16)] = vb >> 3
            a_sub[pl.ds(pos, 16)] = va & 7
            b_sub[pl.ds(pos, 16)] = vb & 7

    lanes = lax.iota(jnp.int32, 16)
    sign_vec = sign_v[...]

    def fire(c, at, bt, sem):
        va = a_rec[pl.ds(c * CHUNK, 16)]
        vb = b_rec[pl.ds(c * CHUNK, 16)]
        for r in range(16):
            pltpu.async_copy(emb_hbm.at[va[r]], at.at[r], sem)
            pltpu.async_copy(ctx_hbm.at[vb[r]], bt.at[r], sem)

    def drain(at, bt, sem):
        for r in range(16):
            pltpu.make_async_copy(emb_hbm.at[0], at.at[r], sem).wait()
            pltpu.make_async_copy(ctx_hbm.at[0], bt.at[r], sem).wait()

    def compute(c, at, bt):
        pos = c * CHUNK
        sub_a = a_sub[pl.ds(pos, 16)]
        sub_b = b_sub[pl.ds(pos, 16)]
        acc = jnp.zeros((16,), jnp.float32)
        for d in range(EMBED):
            d_vec = jnp.full((16,), d, jnp.int32)
            av = plsc.load_gather(at, [lanes, sub_a, d_vec])
            bv = plsc.load_gather(bt, [lanes, sub_b, d_vec])
            acc = acc + av * bv
        z = -(sign_vec * acc)
        loss = jnp.maximum(z, 0.0) + _log1p_of_exp_neg(jnp.abs(z))
        out_v[pl.ds(pos, 16)] = loss

    fire(0, a_t0, b_t0, sem0)
    fire(1, a_t1, b_t1, sem1)

    def body(i, carry):
        e = i * 2
        drain(a_t0, b_t0, sem0)
        compute(e, a_t0, b_t0)
        fire(e + 2, a_t0, b_t0, sem0)
        drain(a_t1, b_t1, sem1)
        compute(e + 1, a_t1, b_t1)
        fire(e + 3, a_t1, b_t1, sem1)
        return carry

    lax.fori_loop(0, NCHUNK // 2 - 1, body, 0)

    e = NCHUNK - 2
    drain(a_t0, b_t0, sem0)
    compute(e, a_t0, b_t0)
    drain(a_t1, b_t1, sem1)
    compute(e + 1, a_t1, b_t1)

    pltpu.sync_copy(out_v, out_hbm.at[pl.ds(base, B_PER_W)])


def kernel(a, b, sign, embeddings, context_embeddings):
    a2 = a.astype(jnp.int32).reshape(NUM_WORKERS * IDX_ROWS, 128)
    b2 = b.astype(jnp.int32).reshape(NUM_WORKERS * IDX_ROWS, 128)
    emb3 = embeddings.reshape(NODE // TILE_ROWS, TILE_ROWS, EMBED)
    ctx3 = context_embeddings.reshape(NODE // TILE_ROWS, TILE_ROWS, EMBED)
    sign_vec = jnp.broadcast_to(jnp.asarray(sign, jnp.float32), (16,))

    buf = pltpu.VMEM((CHUNK, TILE_ROWS, EMBED), jnp.float32)
    mesh = plsc.VectorSubcoreMesh(core_axis_name="c", subcore_axis_name="s")
    run = pl.kernel(
        _sc_body,
        out_type=jax.ShapeDtypeStruct((BATCH,), jnp.float32),
        mesh=mesh,
        compiler_params=pltpu.CompilerParams(needs_layout_passes=False),
        scratch_types=[
            pltpu.VMEM((IDX_ROWS, 128), jnp.int32),
            pltpu.VMEM((IDX_ROWS, 128), jnp.int32),
            pltpu.VMEM((B_PER_W,), jnp.int32),
            pltpu.VMEM((B_PER_W,), jnp.int32),
            pltpu.VMEM((B_PER_W,), jnp.int32),
            pltpu.VMEM((B_PER_W,), jnp.int32),
            buf, buf, buf, buf,
            pltpu.VMEM((B_PER_W,), jnp.float32),
            pltpu.VMEM((16,), jnp.float32),
            pltpu.SemaphoreType.DMA,
            pltpu.SemaphoreType.DMA,
        ],
    )
    return run(a2, b2, sign_vec, emb3, ctx3)
